# baseline (speedup 1.0000x reference)
import functools

import jax
import jax.numpy as jnp
from jax import lax
from jax.experimental import pallas as pl
from jax.experimental.pallas import tpu as pltpu
from jax.experimental.pallas import tpu_sc as plsc

_N = 10000
_NPAD = 10112
_RPS = _NPAD // 16
_NC = 2
_NS = 16
_NW = _NC * _NS
_K = 128
_CH = 79
_PT = _CH * _K
_EP = _NW * _PT
_CH0 = 104
_CH1 = 54
_CHM = _CH0

_PREC = None


def _mesh():
    return plsc.VectorSubcoreMesh(core_axis_name="c", subcore_axis_name="s")


def _make_deg_kernel():
    @functools.partial(
        pl.kernel,
        mesh=_mesh(),
        out_type=jax.ShapeDtypeStruct((_NC, _NPAD, 128), jnp.float32),
        scratch_types=[
            pltpu.VMEM((_CH, _K), jnp.int32),
            pltpu.VMEM((_K, 128), jnp.float32),
            pltpu.VMEM_SHARED((_NPAD, 128), jnp.float32),
            pltpu.SemaphoreType.DMA,
        ],
    )
    def deg_kernel(dstp_hbm, ones_hbm, zeros_hbm, out_hbm, dst_v, ones_v, acc,
                   sem):
        c = lax.axis_index("c")
        s = lax.axis_index("s")
        wid = c * _NS + s
        pltpu.sync_copy(dstp_hbm.at[wid], dst_v)
        pltpu.sync_copy(ones_hbm, ones_v)
        r0 = s * _RPS
        pltpu.sync_copy(zeros_hbm.at[pl.ds(r0, _RPS)], acc.at[pl.ds(r0, _RPS)])
        plsc.subcore_barrier()

        @pl.loop(0, _CH)
        def _(j):
            pltpu.async_copy(ones_v, acc.at[dst_v.at[j]], sem, add=True)

            @pl.when(j >= 1)
            def _():
                pltpu.make_async_copy(ones_v, acc.at[dst_v.at[0]], sem).wait()

        pltpu.make_async_copy(ones_v, acc.at[dst_v.at[0]], sem).wait()

        plsc.subcore_barrier()
        pltpu.sync_copy(acc.at[pl.ds(r0, _RPS)], out_hbm.at[c].at[pl.ds(r0, _RPS)])

    return deg_kernel


def _make_agg_kernel(feat):
    @functools.partial(
        pl.kernel,
        mesh=_mesh(),
        out_type=jax.ShapeDtypeStruct((_NC, _NPAD, feat), jnp.float32),
        scratch_types=[
            pltpu.VMEM((_CHM, _K), jnp.int32),
            pltpu.VMEM((_CHM, _K), jnp.int32),
            pltpu.VMEM((32, feat), jnp.float32),
            pltpu.VMEM((32, feat), jnp.float32),
            pltpu.VMEM((32, feat), jnp.float32),
            pltpu.VMEM((32, feat), jnp.float32),
            pltpu.VMEM_SHARED((_NPAD, feat), jnp.float32),
            pltpu.SemaphoreType.DMA,
            pltpu.SemaphoreType.DMA,
            pltpu.SemaphoreType.DMA,
            pltpu.SemaphoreType.DMA,
            pltpu.SemaphoreType.DMA,
            pltpu.SemaphoreType.DMA,
            pltpu.SemaphoreType.DMA,
            pltpu.SemaphoreType.DMA,
        ],
    )
    def agg_kernel(hs_hbm, srcp_hbm, dstp_hbm, zeros_hbm, out_hbm,
                   src_v, dst_v, r0_v, r1_v, r2_v, r3_v, acc,
                   sm0, sm1, sm2, sm3, ss0, ss1, ss2, ss3):
        rows = (r0_v, r1_v, r2_v, r3_v)
        sems = (sm0, sm1, sm2, sm3)
        ssems = (ss0, ss1, ss2, ss3)
        c = lax.axis_index("c")
        s = lax.axis_index("s")
        wid = c * _NS + s
        pltpu.sync_copy(srcp_hbm.at[wid], src_v)
        pltpu.sync_copy(dstp_hbm.at[wid], dst_v)
        r0 = s * _RPS
        pltpu.sync_copy(zeros_hbm.at[pl.ds(r0, _RPS)], acc.at[pl.ds(r0, _RPS)])
        plsc.subcore_barrier()

        def _drain_scatter(b):
            pltpu.make_async_copy(rows[b], acc.at[dst_v.at[0, pl.ds(0, 32)]],
                                  ssems[b]).wait()

        def edge_loop(ch):
            for q in range(3):
                pltpu.async_copy(hs_hbm.at[src_v.at[0, pl.ds(32 * q, 32)]],
                                 rows[q], sems[q])

            @pl.loop(0, ch)
            def _(j):
                for q in range(4):
                    pltpu.make_async_copy(hs_hbm.at[src_v.at[0, pl.ds(0, 32)]],
                                          rows[q], sems[q]).wait()
                    pltpu.async_copy(rows[q],
                                     acc.at[dst_v.at[j, pl.ds(32 * q, 32)]],
                                     ssems[q], add=True)
                    nq = q + 3
                    tgt = nq % 4
                    if nq < 4:
                        @pl.when(j > 0)
                        def _():
                            _drain_scatter(tgt)

                        pltpu.async_copy(
                            hs_hbm.at[src_v.at[j, pl.ds(32 * nq, 32)]],
                            rows[tgt], sems[tgt])
                    else:
                        _drain_scatter(tgt)

                        @pl.when(j + 1 < ch)
                        def _():
                            pltpu.async_copy(
                                hs_hbm.at[src_v.at[j + 1,
                                                   pl.ds(32 * (nq - 4), 32)]],
                                rows[tgt], sems[tgt])

            _drain_scatter(3)

        @pl.when(c == 0)
        def _():
            edge_loop(_CH0)

        @pl.when(c == 1)
        def _():
            edge_loop(_CH1)

        plsc.subcore_barrier()
        pltpu.sync_copy(acc.at[pl.ds(r0, _RPS)], out_hbm.at[c].at[pl.ds(r0, _RPS)])

    return agg_kernel


_BLK = 1000


def _tc_prep(x_ref, w1_ref, degp_ref, dis_ref, hs1_ref):
    degp = degp_ref[...]
    deg = 1.0 + degp[0, :, 0:1] + degp[1, :, 0:1]
    dis = lax.rsqrt(deg)
    dis_ref[...] = dis
    h = jnp.dot(x_ref[...], w1_ref[...], preferred_element_type=jnp.float32,
                precision=_PREC)
    hs1_ref[...] = h * dis


def _make_tc_mid(f_in, f_out):
    def _tc_mid(sp_ref, hs_ref, dis_ref, b_ref, w_ref, out_ref):
        ssum = sp_ref[0, :, :f_in] + sp_ref[1, :, :f_in]
        dis = dis_ref[...]
        h = jnp.maximum(dis * (ssum + hs_ref[:, :f_in]) + b_ref[...], 0.0)
        hw = jnp.dot(h, w_ref[...], preferred_element_type=jnp.float32,
                     precision=_PREC) * dis
        out_ref[...] = jnp.concatenate(
            [hw, jnp.zeros((_BLK, 128 - f_out), jnp.float32)], axis=1)
    return _tc_mid


def _tc_heads(sp_ref, hs_ref, dis_ref, b3_ref, wt1_ref, bt1_ref, wt2_ref,
              bt2_ref, wp_ref, bp_ref, pe_ref, emb_ref, logit_ref, score_ref):
    ssum = sp_ref[0, :, :32] + sp_ref[1, :, :32]
    dis = dis_ref[...]
    emb = jnp.maximum(dis * (ssum + hs_ref[:, :32]) + b3_ref[...], 0.0)
    emb_ref[...] = emb
    t = jnp.maximum(
        jnp.dot(emb, wt1_ref[...], preferred_element_type=jnp.float32,
                precision=_PREC) + bt1_ref[...], 0.0)
    logit_ref[...] = jnp.dot(t, wt2_ref[...], preferred_element_type=jnp.float32,
                             precision=_PREC) + bt2_ref[...]
    proj = jnp.dot(emb, wp_ref[...], preferred_element_type=jnp.float32,
                   precision=_PREC) + bp_ref[...]
    score_ref[...] = lax.dot_general(
        proj, pe_ref[...], (((1,), (1,)), ((), ())),
        preferred_element_type=jnp.float32, precision=_PREC)


def kernel(x, edge_index, W1, b1, W2, b2, W3, b3, Wt1, bt1, Wt2, bt2, Wp, bp,
           pos_emb):
    src = edge_index[0].astype(jnp.int32)
    dst = edge_index[1].astype(jnp.int32)
    n_edges = src.shape[0]
    pad = _EP - n_edges
    src_f = jnp.concatenate([src, jnp.zeros((pad,), jnp.int32)])
    dst_f = jnp.concatenate([dst, jnp.full((pad,), _N, jnp.int32)])
    dstp = dst_f.reshape(_NW, _CH, _K)

    pt0, pt1 = _CH0 * _K, _CH1 * _K
    padw = _CHM * _K

    def _weighted(flat, fill):
        b0 = flat[:_NS * pt0].reshape(_NS, pt0)
        b1 = flat[_NS * pt0:].reshape(_NS, pt1)
        b0 = jnp.pad(b0, ((0, 0), (0, padw - pt0)), constant_values=fill)
        b1 = jnp.pad(b1, ((0, 0), (0, padw - pt1)), constant_values=fill)
        return jnp.concatenate([b0, b1]).reshape(_NW, _CHM, _K)

    srcp = _weighted(src_f, 0)
    dstp_w = _weighted(dst_f, _N)

    zeros128 = jnp.zeros((_NPAD, 128), jnp.float32)
    ones128 = jnp.ones((_K, 128), jnp.float32)
    degp = _make_deg_kernel()(dstp, ones128, zeros128)

    f32 = jnp.float32
    grid = (_N // _BLK,)
    row = lambda shape: pl.BlockSpec(shape, lambda j: (j,) + (0,) * (len(shape) - 1))
    part = lambda f: pl.BlockSpec((2, _BLK, f), lambda j: (0, j, 0))
    full = lambda shape: pl.BlockSpec(shape, lambda j: (0,) * len(shape))

    dis, hs1 = pl.pallas_call(
        _tc_prep, grid=grid,
        in_specs=[row((_BLK, 128)), full((128, 128)), part(128)],
        out_specs=[row((_BLK, 1)), row((_BLK, 128))],
        out_shape=[jax.ShapeDtypeStruct((_N, 1), f32),
                   jax.ShapeDtypeStruct((_N, 128), f32)],
    )(x, W1, degp)

    agg = _make_agg_kernel(128)

    def mid(f_in, f_out, sp, hs, b, w):
        return pl.pallas_call(
            _make_tc_mid(f_in, f_out), grid=grid,
            in_specs=[part(128), row((_BLK, 128)), row((_BLK, 1)),
                      full((1, f_in)), full((f_in, f_out))],
            out_specs=row((_BLK, 128)),
            out_shape=jax.ShapeDtypeStruct((_N, 128), f32),
        )(sp, hs, dis, b.reshape(1, -1), w)

    s1 = agg(hs1, srcp, dstp_w, zeros128)
    hs2 = mid(128, 64, s1, hs1, b1, W2)

    s2 = agg(hs2, srcp, dstp_w, zeros128)
    hs3 = mid(64, 32, s2, hs2, b2, W3)

    s3 = agg(hs3, srcp, dstp_w, zeros128)
    emb, logit, score = pl.pallas_call(
        _tc_heads, grid=grid,
        in_specs=[part(128), row((_BLK, 128)), row((_BLK, 1)),
                  full((1, 32)), full((32, 16)), full((1, 16)),
                  full((16, 1)), full((1, 1)), full((32, 32)),
                  full((1, 32)), full((7, 32))],
        out_specs=[row((_BLK, 32)), row((_BLK, 1)), row((_BLK, 7))],
        out_shape=[jax.ShapeDtypeStruct((_N, 32), f32),
                   jax.ShapeDtypeStruct((_N, 1), f32),
                   jax.ShapeDtypeStruct((_N, 7), f32)],
    )(s3, hs3, dis, b3.reshape(1, -1), Wt1, bt1.reshape(1, -1), Wt2,
      bt2.reshape(1, -1), Wp, bp.reshape(1, -1), pos_emb)

    return emb, logit[:, 0], score


# speedup vs baseline: 1.0028x; 1.0028x over previous
import functools

import jax
import jax.numpy as jnp
from jax import lax
from jax.experimental import pallas as pl
from jax.experimental.pallas import tpu as pltpu
from jax.experimental.pallas import tpu_sc as plsc

_N = 10000
_NPAD = 10112
_RPS = _NPAD // 16
_NC = 2
_NS = 16
_NW = _NC * _NS
_K = 128
_CH = 79
_PT = _CH * _K
_EP = _NW * _PT
_CH0 = 104
_CH1 = 54
_CHM = _CH0

_PREC = None


def _mesh():
    return plsc.VectorSubcoreMesh(core_axis_name="c", subcore_axis_name="s")


def _make_deg_kernel():
    @functools.partial(
        pl.kernel,
        mesh=_mesh(),
        out_type=jax.ShapeDtypeStruct((_NC, _NPAD, 128), jnp.float32),
        scratch_types=[
            pltpu.VMEM((_CH, _K), jnp.int32),
            pltpu.VMEM((_K, 128), jnp.float32),
            pltpu.VMEM_SHARED((_NPAD, 128), jnp.float32),
        ],
    )
    def deg_kernel(dstp_hbm, ones_hbm, zeros_hbm, out_hbm, dst_v, ones_v, acc):
        c = lax.axis_index("c")
        s = lax.axis_index("s")
        wid = c * _NS + s
        pltpu.sync_copy(dstp_hbm.at[wid], dst_v)
        pltpu.sync_copy(ones_hbm, ones_v)
        r0 = s * _RPS
        pltpu.sync_copy(zeros_hbm.at[pl.ds(r0, _RPS)], acc.at[pl.ds(r0, _RPS)])
        plsc.subcore_barrier()

        @pl.loop(0, _CH)
        def _(j):
            pltpu.sync_copy(ones_v, acc.at[dst_v.at[j]], add=True)

        plsc.subcore_barrier()
        pltpu.sync_copy(acc.at[pl.ds(r0, _RPS)], out_hbm.at[c].at[pl.ds(r0, _RPS)])

    return deg_kernel


def _make_agg_kernel(feat):
    @functools.partial(
        pl.kernel,
        mesh=_mesh(),
        out_type=jax.ShapeDtypeStruct((_NC, _NPAD, feat), jnp.float32),
        scratch_types=[
            pltpu.VMEM((_CHM, _K), jnp.int32),
            pltpu.VMEM((_CHM, _K), jnp.int32),
            pltpu.VMEM((32, feat), jnp.float32),
            pltpu.VMEM((32, feat), jnp.float32),
            pltpu.VMEM((32, feat), jnp.float32),
            pltpu.VMEM((32, feat), jnp.float32),
            pltpu.VMEM_SHARED((_NPAD, feat), jnp.float32),
            pltpu.SemaphoreType.DMA,
            pltpu.SemaphoreType.DMA,
            pltpu.SemaphoreType.DMA,
            pltpu.SemaphoreType.DMA,
        ],
    )
    def agg_kernel(hs_hbm, srcp_hbm, dstp_hbm, zeros_hbm, out_hbm,
                   src_v, dst_v, r0_v, r1_v, r2_v, r3_v, acc,
                   sm0, sm1, sm2, sm3):
        rows = (r0_v, r1_v, r2_v, r3_v)
        sems = (sm0, sm1, sm2, sm3)
        c = lax.axis_index("c")
        s = lax.axis_index("s")
        wid = c * _NS + s
        pltpu.sync_copy(srcp_hbm.at[wid], src_v)
        pltpu.sync_copy(dstp_hbm.at[wid], dst_v)
        r0 = s * _RPS
        pltpu.sync_copy(zeros_hbm.at[pl.ds(r0, _RPS)], acc.at[pl.ds(r0, _RPS)])
        plsc.subcore_barrier()

        def edge_loop(ch):
            for q in range(3):
                pltpu.async_copy(hs_hbm.at[src_v.at[0, pl.ds(32 * q, 32)]],
                                 rows[q], sems[q])

            @pl.loop(0, ch)
            def _(j):
                for q in range(4):
                    pltpu.make_async_copy(hs_hbm.at[src_v.at[0, pl.ds(0, 32)]],
                                          rows[q], sems[q]).wait()
                    nq = q + 3
                    tgt = nq % 4
                    if nq < 4:
                        pltpu.async_copy(
                            hs_hbm.at[src_v.at[j, pl.ds(32 * nq, 32)]],
                            rows[tgt], sems[tgt])
                    else:
                        @pl.when(j + 1 < ch)
                        def _():
                            pltpu.async_copy(
                                hs_hbm.at[src_v.at[j + 1,
                                                   pl.ds(32 * (nq - 4), 32)]],
                                rows[tgt], sems[tgt])

                    pltpu.sync_copy(rows[q],
                                    acc.at[dst_v.at[j, pl.ds(32 * q, 32)]],
                                    add=True)

        @pl.when(c == 0)
        def _():
            edge_loop(_CH0)

        @pl.when(c == 1)
        def _():
            edge_loop(_CH1)

        plsc.subcore_barrier()
        pltpu.sync_copy(acc.at[pl.ds(r0, _RPS)], out_hbm.at[c].at[pl.ds(r0, _RPS)])

    return agg_kernel


_BLK = 1000


def _tc_prep(x_ref, w1_ref, degp_ref, dis_ref, hs1_ref):
    degp = degp_ref[...]
    deg = 1.0 + degp[0, :, 0:1] + degp[1, :, 0:1]
    dis = lax.rsqrt(deg)
    dis_ref[...] = dis
    h = jnp.dot(x_ref[...], w1_ref[...], preferred_element_type=jnp.float32,
                precision=_PREC)
    hs1_ref[...] = h * dis


def _make_tc_mid(f_in, f_out):
    def _tc_mid(sp_ref, hs_ref, dis_ref, b_ref, w_ref, out_ref):
        ssum = sp_ref[0, :, :f_in] + sp_ref[1, :, :f_in]
        dis = dis_ref[...]
        h = jnp.maximum(dis * (ssum + hs_ref[:, :f_in]) + b_ref[...], 0.0)
        hw = jnp.dot(h, w_ref[...], preferred_element_type=jnp.float32,
                     precision=_PREC) * dis
        out_ref[...] = jnp.concatenate(
            [hw, jnp.zeros((_BLK, 128 - f_out), jnp.float32)], axis=1)
    return _tc_mid


def _tc_heads(sp_ref, hs_ref, dis_ref, b3_ref, wt1_ref, bt1_ref, wt2_ref,
              bt2_ref, wp_ref, bp_ref, pe_ref, emb_ref, logit_ref, score_ref):
    ssum = sp_ref[0, :, :32] + sp_ref[1, :, :32]
    dis = dis_ref[...]
    emb = jnp.maximum(dis * (ssum + hs_ref[:, :32]) + b3_ref[...], 0.0)
    emb_ref[...] = emb
    t = jnp.maximum(
        jnp.dot(emb, wt1_ref[...], preferred_element_type=jnp.float32,
                precision=_PREC) + bt1_ref[...], 0.0)
    logit_ref[...] = jnp.dot(t, wt2_ref[...], preferred_element_type=jnp.float32,
                             precision=_PREC) + bt2_ref[...]
    proj = jnp.dot(emb, wp_ref[...], preferred_element_type=jnp.float32,
                   precision=_PREC) + bp_ref[...]
    score_ref[...] = lax.dot_general(
        proj, pe_ref[...], (((1,), (1,)), ((), ())),
        preferred_element_type=jnp.float32, precision=_PREC)


def kernel(x, edge_index, W1, b1, W2, b2, W3, b3, Wt1, bt1, Wt2, bt2, Wp, bp,
           pos_emb):
    src = edge_index[0].astype(jnp.int32)
    dst = edge_index[1].astype(jnp.int32)
    n_edges = src.shape[0]
    pad = _EP - n_edges
    src_f = jnp.concatenate([src, jnp.zeros((pad,), jnp.int32)])
    dst_f = jnp.concatenate([dst, jnp.full((pad,), _N, jnp.int32)])
    dstp = dst_f.reshape(_NW, _CH, _K)

    pt0, pt1 = _CH0 * _K, _CH1 * _K
    padw = _CHM * _K

    def _weighted(flat, fill):
        b0 = flat[:_NS * pt0].reshape(_NS, pt0)
        b1 = flat[_NS * pt0:].reshape(_NS, pt1)
        b0 = jnp.pad(b0, ((0, 0), (0, padw - pt0)), constant_values=fill)
        b1 = jnp.pad(b1, ((0, 0), (0, padw - pt1)), constant_values=fill)
        return jnp.concatenate([b0, b1]).reshape(_NW, _CHM, _K)

    srcp = _weighted(src_f, 0)
    dstp_w = _weighted(dst_f, _N)

    zeros128 = jnp.zeros((_NPAD, 128), jnp.float32)
    ones128 = jnp.ones((_K, 128), jnp.float32)
    degp = _make_deg_kernel()(dstp, ones128, zeros128)

    f32 = jnp.float32
    grid = (_N // _BLK,)
    row = lambda shape: pl.BlockSpec(shape, lambda j: (j,) + (0,) * (len(shape) - 1))
    part = lambda f: pl.BlockSpec((2, _BLK, f), lambda j: (0, j, 0))
    full = lambda shape: pl.BlockSpec(shape, lambda j: (0,) * len(shape))

    dis, hs1 = pl.pallas_call(
        _tc_prep, grid=grid,
        in_specs=[row((_BLK, 128)), full((128, 128)), part(128)],
        out_specs=[row((_BLK, 1)), row((_BLK, 128))],
        out_shape=[jax.ShapeDtypeStruct((_N, 1), f32),
                   jax.ShapeDtypeStruct((_N, 128), f32)],
    )(x, W1, degp)

    agg = _make_agg_kernel(128)

    def mid(f_in, f_out, sp, hs, b, w):
        return pl.pallas_call(
            _make_tc_mid(f_in, f_out), grid=grid,
            in_specs=[part(128), row((_BLK, 128)), row((_BLK, 1)),
                      full((1, f_in)), full((f_in, f_out))],
            out_specs=row((_BLK, 128)),
            out_shape=jax.ShapeDtypeStruct((_N, 128), f32),
        )(sp, hs, dis, b.reshape(1, -1), w)

    s1 = agg(hs1, srcp, dstp_w, zeros128)
    hs2 = mid(128, 64, s1, hs1, b1, W2)

    s2 = agg(hs2, srcp, dstp_w, zeros128)
    hs3 = mid(64, 32, s2, hs2, b2, W3)

    s3 = agg(hs3, srcp, dstp_w, zeros128)
    emb, logit, score = pl.pallas_call(
        _tc_heads, grid=grid,
        in_specs=[part(128), row((_BLK, 128)), row((_BLK, 1)),
                  full((1, 32)), full((32, 16)), full((1, 16)),
                  full((16, 1)), full((1, 1)), full((32, 32)),
                  full((1, 32)), full((7, 32))],
        out_specs=[row((_BLK, 32)), row((_BLK, 1)), row((_BLK, 7))],
        out_shape=[jax.ShapeDtypeStruct((_N, 32), f32),
                   jax.ShapeDtypeStruct((_N, 1), f32),
                   jax.ShapeDtypeStruct((_N, 7), f32)],
    )(s3, hs3, dis, b3.reshape(1, -1), Wt1, bt1.reshape(1, -1), Wt2,
      bt2.reshape(1, -1), Wp, bp.reshape(1, -1), pos_emb)

    return emb, logit[:, 0], score
